# Initial kernel scaffold; baseline (speedup 1.0000x reference)
import functools

import jax
import jax.numpy as jnp
from jax import lax
from jax.experimental import pallas as pl
from jax.experimental.pallas import tpu as pltpu
from jax.experimental.pallas import tpu_sc as plsc

_VS = 100000
_HS = 4096
_HD = 64
_MUL = _VS % _HS
_NC = 2
_NS = 16
_L = 16
_NW = _NC * _NS
_CHUNK = 1024
_NVEC = _CHUNK // _L
_NROW = _CHUNK // 128


def _make_kernel(BS, tok_w, nch):
    mesh = plsc.VectorSubcoreMesh(
        core_axis_name="c", subcore_axis_name="s",
        num_cores=_NC, num_subcores=_NS)

    @functools.partial(
        pl.kernel,
        out_type=jax.ShapeDtypeStruct((BS, 2, _HD), jnp.float32),
        mesh=mesh,
        scratch_types=[
            pltpu.VMEM((_NROW + 1, 128), jnp.int32),
            pltpu.VMEM((_NROW, 128), jnp.int32),
            pltpu.VMEM((128, _HD), jnp.float32),
            pltpu.VMEM((128, _HD), jnp.float32),
            pltpu.SemaphoreType.DMA,
        ],
    )
    def k(ids2d, u_hbm, b_hbm, out, idsv, biv, buf_u, buf_b, sem):
        cid = lax.axis_index("c")
        sid = lax.axis_index("s")
        wid = sid * _NC + cid
        base = wid * tok_w

        def chunk_body(c, carry):
            t0 = base + c * _CHUNK
            g0 = t0 // 128

            @pl.when(g0 > 0)
            def _():
                pltpu.sync_copy(ids2d.at[pl.ds(g0 - 1, _NROW + 1)], idsv)

            @pl.when(g0 == 0)
            def _():
                pltpu.sync_copy(ids2d.at[pl.ds(0, _NROW)],
                                idsv.at[pl.ds(1, _NROW)])

            def vec_body(i, carry2):
                r = 1 + (i >> 3)
                cb = (i & 7) * _L
                ids_vec = idsv[r, pl.ds(cb, _L)]
                kloc = i * _L + lax.iota(jnp.int32, _L)
                kp = kloc + 127
                pi_vec = plsc.load_gather(idsv, [kp >> 7, kp & 127])
                pos = t0 + kloc
                pi_vec = jnp.where(pos % 200 == 0, 0, pi_vec)
                bi = ((pi_vec & (_HS - 1)) * _MUL + (ids_vec & (_HS - 1))) \
                    & (_HS - 1)
                biv[i >> 3, pl.ds(cb, _L)] = bi
                return carry2

            lax.fori_loop(0, _NVEC, vec_body, 0, unroll=2)

            for j in range(_NROW):
                cpu = pltpu.async_copy(u_hbm.at[idsv.at[1 + j]], buf_u, sem)
                cpb = pltpu.async_copy(b_hbm.at[biv.at[j]], buf_b, sem)
                cpu.wait()
                pltpu.sync_copy(buf_u, out.at[pl.ds(t0 + j * 128, 128), 0])
                cpb.wait()
                pltpu.sync_copy(buf_b, out.at[pl.ds(t0 + j * 128, 128), 1])
            return carry

        lax.fori_loop(0, nch, chunk_body, 0)

    return k


def kernel(ids, U, Bt):
    B, S = ids.shape
    BS = B * S
    tok_w = BS // _NW
    nch = tok_w // _CHUNK
    ids2d = ids.reshape(BS // 128, 128)
    out = _make_kernel(BS, tok_w, nch)(ids2d, U, Bt)
    return out.reshape(B, S, 2 * _HD)


# speedup vs baseline: 13.1750x; 13.1750x over previous
import functools

import jax
import jax.numpy as jnp
from jax import lax
from jax.experimental import pallas as pl
from jax.experimental.pallas import tpu as pltpu
from jax.experimental.pallas import tpu_sc as plsc

_VS = 100000
_HS = 4096
_HD = 64
_MUL = _VS % _HS
_NC = 2
_NS = 16
_L = 16
_NW = _NC * _NS
_CHUNK = 1024


def _dyn_gather(x, idx):
    return lax.gather(
        x, idx[:, None],
        lax.GatherDimensionNumbers(
            offset_dims=(), collapsed_slice_dims=(0,), start_index_map=(0,)),
        (1,), mode=lax.GatherScatterMode.PROMISE_IN_BOUNDS)
_NVEC = _CHUNK // _L
_NROW = _CHUNK // 128


def _make_kernel(BS, tok_w, nch):
    mesh = plsc.VectorSubcoreMesh(
        core_axis_name="c", subcore_axis_name="s",
        num_cores=_NC, num_subcores=_NS)

    @functools.partial(
        pl.kernel,
        out_type=jax.ShapeDtypeStruct((BS, 2, _HD), jnp.float32),
        mesh=mesh,
        scratch_types=[
            pltpu.VMEM((_NROW + 8, 128), jnp.int32),
            pltpu.VMEM((_NROW, 128), jnp.int32),
            pltpu.VMEM((128, _HD), jnp.float32),
            pltpu.VMEM((128, _HD), jnp.float32),
            pltpu.SemaphoreType.DMA,
        ],
        compiler_params=pltpu.CompilerParams(use_tc_tiling_on_sc=False),
    )
    def k(ids2d, u_hbm, b_hbm, out, idsv, biv, buf_u, buf_b, sem):
        cid = lax.axis_index("c")
        sid = lax.axis_index("s")
        wid = sid * _NC + cid
        base = wid * tok_w

        def chunk_body(c, carry):
            t0 = base + c * _CHUNK
            g0 = t0 // 128

            @pl.when(g0 > 0)
            def _():
                pltpu.sync_copy(
                    ids2d.at[pl.ds(pl.multiple_of(g0 - 8, 8), _NROW + 8)],
                    idsv)

            @pl.when(g0 == 0)
            def _():
                pltpu.sync_copy(ids2d.at[pl.ds(0, _NROW)],
                                idsv.at[pl.ds(8, _NROW)])

            iota = lax.iota(jnp.int32, _L)
            shift_idx = jnp.maximum(iota - 1, 0)
            lane15 = jnp.full((_L,), 15, jnp.int32)

            def vec_body(i, prev_vec):
                r = 8 + (i >> 3)
                cb = (i & 7) * _L
                ids_vec = idsv[r, pl.ds(cb, _L)]
                shifted = _dyn_gather(ids_vec, shift_idx)
                prev_last = _dyn_gather(prev_vec, lane15)
                pi_vec = jnp.where(iota == 0, prev_last, shifted)
                pos = t0 + i * _L + iota
                pi_vec = jnp.where(pos % 200 == 0, 0, pi_vec)
                bi = ((pi_vec & (_HS - 1)) * _MUL + (ids_vec & (_HS - 1))) \
                    & (_HS - 1)
                biv[i >> 3, pl.ds(cb, _L)] = bi
                return ids_vec

            lax.fori_loop(0, _NVEC, vec_body, idsv[7, pl.ds(112, _L)],
                          unroll=2)

            for j in range(_NROW):
                cpu = pltpu.async_copy(u_hbm.at[idsv.at[8 + j]], buf_u, sem)
                cpb = pltpu.async_copy(b_hbm.at[biv.at[j]], buf_b, sem)
                orow = pl.multiple_of(t0 + j * 128, 128)
                cpu.wait()
                pltpu.sync_copy(buf_u, out.at[pl.ds(orow, 128), 0])
                cpb.wait()
                pltpu.sync_copy(buf_b, out.at[pl.ds(orow, 128), 1])
            return carry

        lax.fori_loop(0, nch, chunk_body, 0)

    return k


def kernel(ids, U, Bt):
    B, S = ids.shape
    BS = B * S
    tok_w = BS // _NW
    nch = tok_w // _CHUNK
    ids2d = ids.reshape(BS // 128, 128)
    out = _make_kernel(BS, tok_w, nch)(ids2d, U, Bt)
    return out.reshape(B, S, 2 * _HD)


# speedup vs baseline: 14.1407x; 1.0733x over previous
import functools

import jax
import jax.numpy as jnp
from jax import lax
from jax.experimental import pallas as pl
from jax.experimental.pallas import tpu as pltpu
from jax.experimental.pallas import tpu_sc as plsc

_VS = 100000
_HS = 4096
_HD = 64
_MUL = _VS % _HS
_NC = 2
_NS = 16
_L = 16
_NW = _NC * _NS
_CHUNK = 1024


def _dyn_gather(x, idx):
    return lax.gather(
        x, idx[:, None],
        lax.GatherDimensionNumbers(
            offset_dims=(), collapsed_slice_dims=(0,), start_index_map=(0,)),
        (1,), mode=lax.GatherScatterMode.PROMISE_IN_BOUNDS)
_NVEC = _CHUNK // _L
_NROW = _CHUNK // 128


def _make_kernel(BS, tok_w, nch):
    mesh = plsc.VectorSubcoreMesh(
        core_axis_name="c", subcore_axis_name="s",
        num_cores=_NC, num_subcores=_NS)

    @functools.partial(
        pl.kernel,
        out_type=jax.ShapeDtypeStruct((BS, 2, _HD), jnp.float32),
        mesh=mesh,
        scratch_types=[
            pltpu.VMEM((_NROW + 8, 128), jnp.int32),
            pltpu.VMEM((_NROW, 128), jnp.int32),
            pltpu.VMEM((2, 128, _HD), jnp.float32),
            pltpu.VMEM((2, 128, _HD), jnp.float32),
            pltpu.SemaphoreType.DMA,
            pltpu.SemaphoreType.DMA,
        ],
        compiler_params=pltpu.CompilerParams(use_tc_tiling_on_sc=False),
    )
    def k(ids2d, u_hbm, b_hbm, out, idsv, biv, buf_u, buf_b, gsem, wsem):
        cid = lax.axis_index("c")
        sid = lax.axis_index("s")
        wid = sid * _NC + cid
        base = wid * tok_w

        def chunk_body(c, carry):
            t0 = base + c * _CHUNK
            g0 = t0 // 128

            @pl.when(g0 > 0)
            def _():
                pltpu.sync_copy(
                    ids2d.at[pl.ds(pl.multiple_of(g0 - 8, 8), _NROW + 8)],
                    idsv)

            @pl.when(g0 == 0)
            def _():
                pltpu.sync_copy(ids2d.at[pl.ds(0, _NROW)],
                                idsv.at[pl.ds(8, _NROW)])

            iota = lax.iota(jnp.int32, _L)
            shift_idx = jnp.maximum(iota - 1, 0)
            lane15 = jnp.full((_L,), 15, jnp.int32)

            def vec_body(i, prev_vec):
                r = 8 + (i >> 3)
                cb = (i & 7) * _L
                ids_vec = idsv[r, pl.ds(cb, _L)]
                shifted = _dyn_gather(ids_vec, shift_idx)
                prev_last = _dyn_gather(prev_vec, lane15)
                pi_vec = jnp.where(iota == 0, prev_last, shifted)
                pos = t0 + i * _L + iota
                pi_vec = jnp.where(pos % 200 == 0, 0, pi_vec)
                bi = ((pi_vec & (_HS - 1)) * _MUL + (ids_vec & (_HS - 1))) \
                    & (_HS - 1)
                biv[i >> 3, pl.ds(cb, _L)] = bi
                return ids_vec

            lax.fori_loop(0, _NVEC, vec_body, idsv[7, pl.ds(112, _L)],
                          unroll=2)

            wpend = [None] * _NROW
            for j in range(_NROW):
                s = j % 2
                if j >= 2:
                    wpend[j - 2][0].wait()
                    wpend[j - 2][1].wait()
                cpu = pltpu.async_copy(u_hbm.at[idsv.at[8 + j]],
                                       buf_u.at[s], gsem)
                cpb = pltpu.async_copy(b_hbm.at[biv.at[j]],
                                       buf_b.at[s], gsem)
                orow = pl.multiple_of(t0 + j * 128, 128)
                cpu.wait()
                cpb.wait()
                wpend[j] = (
                    pltpu.async_copy(buf_u.at[s],
                                     out.at[pl.ds(orow, 128), 0], wsem),
                    pltpu.async_copy(buf_b.at[s],
                                     out.at[pl.ds(orow, 128), 1], wsem),
                )
            for j in (_NROW - 2, _NROW - 1):
                wpend[j][0].wait()
                wpend[j][1].wait()
            return carry

        lax.fori_loop(0, nch, chunk_body, 0)

    return k


def kernel(ids, U, Bt):
    B, S = ids.shape
    BS = B * S
    tok_w = BS // _NW
    nch = tok_w // _CHUNK
    ids2d = ids.reshape(BS // 128, 128)
    out = _make_kernel(BS, tok_w, nch)(ids2d, U, Bt)
    return out.reshape(B, S, 2 * _HD)


# speedup vs baseline: 16.9093x; 1.1958x over previous
import functools

import jax
import jax.numpy as jnp
from jax import lax
from jax.experimental import pallas as pl
from jax.experimental.pallas import tpu as pltpu
from jax.experimental.pallas import tpu_sc as plsc

_VS = 100000
_HS = 4096
_HD = 64
_MUL = _VS % _HS
_NC = 2
_NS = 16
_L = 16
_NW = _NC * _NS
_CHUNK = 1024


def _dyn_gather(x, idx):
    return lax.gather(
        x, idx[:, None],
        lax.GatherDimensionNumbers(
            offset_dims=(), collapsed_slice_dims=(0,), start_index_map=(0,)),
        (1,), mode=lax.GatherScatterMode.PROMISE_IN_BOUNDS)
_NVEC = _CHUNK // _L
_NROW = _CHUNK // 128


def _make_kernel(BS, tok_w, nch):
    mesh = plsc.VectorSubcoreMesh(
        core_axis_name="c", subcore_axis_name="s",
        num_cores=_NC, num_subcores=_NS)

    @functools.partial(
        pl.kernel,
        out_type=jax.ShapeDtypeStruct((BS, 2, _HD), jnp.float32),
        mesh=mesh,
        scratch_types=[
            pltpu.VMEM((_NROW + 8, 128), jnp.int32),
            pltpu.VMEM((_NROW, 128), jnp.int32),
            pltpu.VMEM((4, 128, _HD), jnp.float32),
            pltpu.VMEM((4, 128, _HD), jnp.float32),
            pltpu.SemaphoreType.DMA,
            pltpu.SemaphoreType.DMA,
        ],
        compiler_params=pltpu.CompilerParams(use_tc_tiling_on_sc=False),
    )
    def k(ids2d, u_hbm, b_hbm, out, idsv, biv, buf_u, buf_b, gsem, wsem):
        cid = lax.axis_index("c")
        sid = lax.axis_index("s")
        wid = sid * _NC + cid
        base = wid * tok_w

        def chunk_body(c, carry):
            t0 = base + c * _CHUNK
            g0 = t0 // 128

            @pl.when(g0 > 0)
            def _():
                pltpu.sync_copy(
                    ids2d.at[pl.ds(pl.multiple_of(g0 - 8, 8), _NROW + 8)],
                    idsv)

            @pl.when(g0 == 0)
            def _():
                pltpu.sync_copy(ids2d.at[pl.ds(0, _NROW)],
                                idsv.at[pl.ds(8, _NROW)])

            iota = lax.iota(jnp.int32, _L)
            shift_idx = jnp.maximum(iota - 1, 0)
            lane15 = jnp.full((_L,), 15, jnp.int32)

            def vec_body(i, prev_vec):
                r = 8 + (i >> 3)
                cb = (i & 7) * _L
                ids_vec = idsv[r, pl.ds(cb, _L)]
                shifted = _dyn_gather(ids_vec, shift_idx)
                prev_last = _dyn_gather(prev_vec, lane15)
                pi_vec = jnp.where(iota == 0, prev_last, shifted)
                pos = t0 + i * _L + iota
                pi_vec = jnp.where(pos % 200 == 0, 0, pi_vec)
                bi = ((pi_vec & (_HS - 1)) * _MUL + (ids_vec & (_HS - 1))) \
                    & (_HS - 1)
                biv[i >> 3, pl.ds(cb, _L)] = bi
                return ids_vec

            lax.fori_loop(0, _NVEC, vec_body, idsv[7, pl.ds(112, _L)],
                          unroll=2)

            def fire_gathers(j):
                s = j % 4
                gu = pltpu.async_copy(u_hbm.at[idsv.at[8 + j]],
                                      buf_u.at[s], gsem)
                gb = pltpu.async_copy(b_hbm.at[biv.at[j]],
                                      buf_b.at[s], gsem)
                return gu, gb

            gpend = [None] * _NROW
            wpend = [None] * _NROW
            for j in range(min(2, _NROW)):
                gpend[j] = fire_gathers(j)
            for j in range(_NROW):
                s = j % 4
                if j + 2 < _NROW:
                    if j + 2 >= 4:
                        wpend[j - 2][0].wait()
                        wpend[j - 2][1].wait()
                    gpend[j + 2] = fire_gathers(j + 2)
                gpend[j][0].wait()
                gpend[j][1].wait()
                orow = pl.multiple_of(t0 + j * 128, 128)
                wpend[j] = (
                    pltpu.async_copy(buf_u.at[s],
                                     out.at[pl.ds(orow, 128), 0], wsem),
                    pltpu.async_copy(buf_b.at[s],
                                     out.at[pl.ds(orow, 128), 1], wsem),
                )
            for j in range(_NROW - 4, _NROW):
                wpend[j][0].wait()
                wpend[j][1].wait()
            return carry

        lax.fori_loop(0, nch, chunk_body, 0)

    return k


def kernel(ids, U, Bt):
    B, S = ids.shape
    BS = B * S
    tok_w = BS // _NW
    nch = tok_w // _CHUNK
    ids2d = ids.reshape(BS // 128, 128)
    out = _make_kernel(BS, tok_w, nch)(ids2d, U, Bt)
    return out.reshape(B, S, 2 * _HD)


# speedup vs baseline: 16.9331x; 1.0014x over previous
import functools

import jax
import jax.numpy as jnp
from jax import lax
from jax.experimental import pallas as pl
from jax.experimental.pallas import tpu as pltpu
from jax.experimental.pallas import tpu_sc as plsc

_VS = 100000
_HS = 4096
_HD = 64
_MUL = _VS % _HS
_NC = 2
_NS = 16
_L = 16
_NW = _NC * _NS
_CHUNK = 1024
_RING = 6
_AHEAD = 3


def _dyn_gather(x, idx):
    return lax.gather(
        x, idx[:, None],
        lax.GatherDimensionNumbers(
            offset_dims=(), collapsed_slice_dims=(0,), start_index_map=(0,)),
        (1,), mode=lax.GatherScatterMode.PROMISE_IN_BOUNDS)
_NVEC = _CHUNK // _L
_NROW = _CHUNK // 128


def _make_kernel(BS, tok_w, nch):
    mesh = plsc.VectorSubcoreMesh(
        core_axis_name="c", subcore_axis_name="s",
        num_cores=_NC, num_subcores=_NS)

    @functools.partial(
        pl.kernel,
        out_type=jax.ShapeDtypeStruct((BS, 2, _HD), jnp.float32),
        mesh=mesh,
        scratch_types=[
            pltpu.VMEM((_NROW + 8, 128), jnp.int32),
            pltpu.VMEM((_NROW, 128), jnp.int32),
            pltpu.VMEM((6, 128, _HD), jnp.float32),
            pltpu.VMEM((6, 128, _HD), jnp.float32),
            pltpu.SemaphoreType.DMA,
            pltpu.SemaphoreType.DMA,
        ],
        compiler_params=pltpu.CompilerParams(use_tc_tiling_on_sc=False),
    )
    def k(ids2d, u_hbm, b_hbm, out, idsv, biv, buf_u, buf_b, gsem, wsem):
        cid = lax.axis_index("c")
        sid = lax.axis_index("s")
        wid = sid * _NC + cid
        base = wid * tok_w

        def chunk_body(c, carry):
            t0 = base + c * _CHUNK
            g0 = t0 // 128

            @pl.when(g0 > 0)
            def _():
                pltpu.sync_copy(
                    ids2d.at[pl.ds(pl.multiple_of(g0 - 8, 8), _NROW + 8)],
                    idsv)

            @pl.when(g0 == 0)
            def _():
                pltpu.sync_copy(ids2d.at[pl.ds(0, _NROW)],
                                idsv.at[pl.ds(8, _NROW)])

            iota = lax.iota(jnp.int32, _L)
            shift_idx = jnp.maximum(iota - 1, 0)
            lane15 = jnp.full((_L,), 15, jnp.int32)

            def vec_body(i, prev_vec):
                r = 8 + (i >> 3)
                cb = (i & 7) * _L
                ids_vec = idsv[r, pl.ds(cb, _L)]
                shifted = _dyn_gather(ids_vec, shift_idx)
                prev_last = _dyn_gather(prev_vec, lane15)
                pi_vec = jnp.where(iota == 0, prev_last, shifted)
                pos = t0 + i * _L + iota
                pi_vec = jnp.where(pos % 200 == 0, 0, pi_vec)
                bi = ((pi_vec & (_HS - 1)) * _MUL + (ids_vec & (_HS - 1))) \
                    & (_HS - 1)
                biv[i >> 3, pl.ds(cb, _L)] = bi
                return ids_vec

            lax.fori_loop(0, _NVEC, vec_body, idsv[7, pl.ds(112, _L)],
                          unroll=2)

            def fire_gathers(j):
                s = j % _RING
                gu = pltpu.async_copy(u_hbm.at[idsv.at[8 + j]],
                                      buf_u.at[s], gsem)
                gb = pltpu.async_copy(b_hbm.at[biv.at[j]],
                                      buf_b.at[s], gsem)
                return gu, gb

            gpend = [None] * _NROW
            wpend = [None] * _NROW
            for j in range(min(_AHEAD, _NROW)):
                gpend[j] = fire_gathers(j)
            for j in range(_NROW):
                s = j % _RING
                if j + _AHEAD < _NROW:
                    if j + _AHEAD >= _RING:
                        wpend[j + _AHEAD - _RING][0].wait()
                        wpend[j + _AHEAD - _RING][1].wait()
                        wpend[j + _AHEAD - _RING] = None
                    gpend[j + _AHEAD] = fire_gathers(j + _AHEAD)
                gpend[j][0].wait()
                gpend[j][1].wait()
                orow = pl.multiple_of(t0 + j * 128, 128)
                wpend[j] = (
                    pltpu.async_copy(buf_u.at[s],
                                     out.at[pl.ds(orow, 128), 0], wsem),
                    pltpu.async_copy(buf_b.at[s],
                                     out.at[pl.ds(orow, 128), 1], wsem),
                )
            for j in range(_NROW):
                if wpend[j] is not None:
                    wpend[j][0].wait()
                    wpend[j][1].wait()
                    wpend[j] = None
            return carry

        lax.fori_loop(0, nch, chunk_body, 0)

    return k


def kernel(ids, U, Bt):
    B, S = ids.shape
    BS = B * S
    tok_w = BS // _NW
    nch = tok_w // _CHUNK
    ids2d = ids.reshape(BS // 128, 128)
    out = _make_kernel(BS, tok_w, nch)(ids2d, U, Bt)
    return out.reshape(B, S, 2 * _HD)


# speedup vs baseline: 17.0127x; 1.0047x over previous
import functools

import jax
import jax.numpy as jnp
from jax import lax
from jax.experimental import pallas as pl
from jax.experimental.pallas import tpu as pltpu
from jax.experimental.pallas import tpu_sc as plsc

_VS = 100000
_HS = 4096
_HD = 64
_MUL = _VS % _HS
_NC = 2
_NS = 16
_L = 16
_NW = _NC * _NS
_CHUNK = 1024
_NVEC = _CHUNK // _L
_NROW = _CHUNK // 128
_VPR = 128 // _L
_RING = 6
_AHEAD = 3


def _dyn_gather(x, idx):
    return lax.gather(
        x, idx[:, None],
        lax.GatherDimensionNumbers(
            offset_dims=(), collapsed_slice_dims=(0,), start_index_map=(0,)),
        (1,), mode=lax.GatherScatterMode.PROMISE_IN_BOUNDS)


def _make_kernel(BS, tok_w, nch):
    mesh = plsc.VectorSubcoreMesh(
        core_axis_name="c", subcore_axis_name="s",
        num_cores=_NC, num_subcores=_NS)

    @functools.partial(
        pl.kernel,
        out_type=jax.ShapeDtypeStruct((BS, 2, _HD), jnp.float32),
        mesh=mesh,
        scratch_types=[
            pltpu.VMEM((2, _NROW + 8, 128), jnp.int32),
            pltpu.VMEM((2, _NROW, 128), jnp.int32),
            pltpu.VMEM((_RING, 128, _HD), jnp.float32),
            pltpu.VMEM((_RING, 128, _HD), jnp.float32),
            pltpu.SemaphoreType.DMA,
            pltpu.SemaphoreType.DMA,
        ],
        compiler_params=pltpu.CompilerParams(use_tc_tiling_on_sc=False),
    )
    def k(ids2d, u_hbm, b_hbm, out, idsv, biv, buf_u, buf_b, gsem, wsem):
        cid = lax.axis_index("c")
        sid = lax.axis_index("s")
        wid = sid * _NC + cid
        base = wid * tok_w

        iota = lax.iota(jnp.int32, _L)
        shift_idx = jnp.maximum(iota - 1, 0)
        lane15 = jnp.full((_L,), 15, jnp.int32)

        def hash_vec(q, i, r, cb, t0n, prev_vec):
            ids_vec = idsv[q, r, pl.ds(cb, _L)]
            shifted = _dyn_gather(ids_vec, shift_idx)
            prev_last = _dyn_gather(prev_vec, lane15)
            pi_vec = jnp.where(iota == 0, prev_last, shifted)
            pos = t0n + i * _L + iota
            pi_vec = jnp.where(pos % 200 == 0, 0, pi_vec)
            bi = ((pi_vec & (_HS - 1)) * _MUL + (ids_vec & (_HS - 1))) \
                & (_HS - 1)
            return ids_vec, bi

        def stage_chunk(q, g0):
            @pl.when(g0 > 0)
            def _():
                pltpu.sync_copy(
                    ids2d.at[pl.ds(pl.multiple_of(g0 - 8, 8), _NROW + 8)],
                    idsv.at[q])

            @pl.when(g0 == 0)
            def _():
                pltpu.sync_copy(ids2d.at[pl.ds(0, _NROW)],
                                idsv.at[q, pl.ds(8, _NROW)])

        stage_chunk(0, base // 128)

        def pro_body(i, prev_vec):
            r = 8 + (i >> 3)
            cb = (i & 7) * _L
            ids_vec, bi = hash_vec(0, i, r, cb, base, prev_vec)
            biv[0, i >> 3, pl.ds(cb, _L)] = bi
            return ids_vec

        lax.fori_loop(0, _NVEC, pro_body, idsv[0, 7, pl.ds(112, _L)],
                      unroll=2)

        def chunk_body(c, carry):
            p = c & 1
            q = 1 - p
            t0 = base + c * _CHUNK
            t0n = t0 + _CHUNK
            cn_ok = c + 1 < nch

            def fire_gathers(j):
                s = j % _RING
                gu = pltpu.async_copy(u_hbm.at[idsv.at[p, 8 + j]],
                                      buf_u.at[s], gsem)
                gb = pltpu.async_copy(b_hbm.at[biv.at[p, j]],
                                      buf_b.at[s], gsem)
                return gu, gb

            gpend = [None] * _NROW
            wpend = [None] * _NROW
            for j in range(min(_AHEAD, _NROW)):
                gpend[j] = fire_gathers(j)

            @pl.when(cn_ok)
            def _():
                g0n = t0n // 128
                pltpu.sync_copy(
                    ids2d.at[pl.ds(pl.multiple_of(g0n - 8, 8), _NROW + 8)],
                    idsv.at[q])

            for j in range(_NROW):
                s = j % _RING
                if j + _AHEAD < _NROW:
                    if j + _AHEAD >= _RING:
                        wpend[j + _AHEAD - _RING][0].wait()
                        wpend[j + _AHEAD - _RING][1].wait()
                        wpend[j + _AHEAD - _RING] = None
                    gpend[j + _AHEAD] = fire_gathers(j + _AHEAD)

                @pl.when(cn_ok)
                def _(j=j):
                    i0 = j * _VPR
                    rp = 8 + ((i0 - 1) >> 3)
                    cp = ((i0 - 1) & 7) * _L
                    prev = idsv[q, rp, pl.ds(cp, _L)]
                    for i in range(i0, i0 + _VPR):
                        r = 8 + (i >> 3)
                        cb = (i & 7) * _L
                        prev, bi = hash_vec(q, i, r, cb, t0n, prev)
                        biv[q, j, pl.ds(cb, _L)] = bi

                gpend[j][0].wait()
                gpend[j][1].wait()
                orow = pl.multiple_of(t0 + j * 128, 128)
                wpend[j] = (
                    pltpu.async_copy(buf_u.at[s],
                                     out.at[pl.ds(orow, 128), 0], wsem),
                    pltpu.async_copy(buf_b.at[s],
                                     out.at[pl.ds(orow, 128), 1], wsem),
                )
            for j in range(_NROW):
                if wpend[j] is not None:
                    wpend[j][0].wait()
                    wpend[j][1].wait()
                    wpend[j] = None
            return carry

        lax.fori_loop(0, nch, chunk_body, 0)

    return k


def kernel(ids, U, Bt):
    B, S = ids.shape
    BS = B * S
    tok_w = BS // _NW
    nch = tok_w // _CHUNK
    ids2d = ids.reshape(BS // 128, 128)
    out = _make_kernel(BS, tok_w, nch)(ids2d, U, Bt)
    return out.reshape(B, S, 2 * _HD)
